# Initial kernel scaffold; baseline (speedup 1.0000x reference)
import jax
import jax.numpy as jnp
from jax.experimental import pallas as pl


def kernel(inputs, labels):
    raise NotImplementedError("write your pallas kernel here")


# speedup vs baseline: 1.0168x; 1.0168x over previous
import functools

import jax
import jax.numpy as jnp
from jax import lax
from jax.experimental import pallas as pl
from jax.experimental.pallas import tpu as pltpu
from jax.experimental.pallas import tpu_sc as plsc

N = 16384
C = 1000
CPAD = 1024
BR = 256
NW = 16
EPW = N // NW
RROWS = CPAD // 16


def _losses_body(x_ref, lab_ref, out_ref):
    x = x_ref[...]
    lab = lab_ref[...]
    m = jnp.max(x, axis=1)
    e = jnp.exp(x - m[:, None])
    s = jnp.sum(e, axis=1)
    logz = m + jnp.log(s)
    col = lax.broadcasted_iota(jnp.int32, x.shape, 1)
    picked = jnp.sum(jnp.where(col == lab[:, None], x, 0.0), axis=1)
    out_ref[...] = logz - picked


def _losses(inputs, labels):
    return pl.pallas_call(
        _losses_body,
        grid=(N // BR,),
        in_specs=[
            pl.BlockSpec((BR, C), lambda i: (i, 0)),
            pl.BlockSpec((BR,), lambda i: (i,)),
        ],
        out_specs=pl.BlockSpec((BR,), lambda i: (i,)),
        out_shape=jax.ShapeDtypeStruct((N,), jnp.float32),
    )(inputs, labels)


def _groupby(losses, labels):
    mesh = plsc.VectorSubcoreMesh(
        core_axis_name="c", subcore_axis_name="s", num_cores=1)

    @functools.partial(
        pl.kernel,
        mesh=mesh,
        compiler_params=pltpu.CompilerParams(
            use_tc_tiling_on_sc=False, needs_layout_passes=False),
        out_type=[
            jax.ShapeDtypeStruct((RROWS, 16), jnp.float32),
            jax.ShapeDtypeStruct((RROWS, 16), jnp.float32),
        ],
        scratch_types=[
            pltpu.VMEM((EPW,), jnp.int32),
            pltpu.VMEM((EPW,), jnp.float32),
            pltpu.VMEM((16 * CPAD,), jnp.float32),
            pltpu.VMEM((16 * CPAD,), jnp.float32),
            pltpu.VMEM((RROWS, 16), jnp.float32),
            pltpu.VMEM((RROWS, 16), jnp.float32),
            pltpu.VMEM((RROWS,), jnp.int32),
            pltpu.VMEM_SHARED((RROWS, 16), jnp.float32),
            pltpu.VMEM_SHARED((RROWS, 16), jnp.float32),
        ],
    )
    def k(loss_hbm, lab_hbm, sum_hbm, cnt_hbm,
          lab_v, loss_v, bins, cbins, red_s, red_c, idx_v, sh_s, sh_c):
        wid = lax.axis_index("s")
        lane = lax.broadcasted_iota(jnp.int32, (16,), 0)
        zeros = jnp.zeros((16,), jnp.float32)
        ones = jnp.ones((16,), jnp.float32)

        def zero_body(j, _):
            bins[pl.ds(j * 16, 16)] = zeros
            cbins[pl.ds(j * 16, 16)] = zeros
            return 0
        lax.fori_loop(0, 16 * CPAD // 16, zero_body, 0)
        for j in range(RROWS):
            red_s[j, :] = zeros
            red_c[j, :] = zeros
        for j in range(RROWS // 16):
            idx_v[pl.ds(j * 16, 16)] = lane + (j * 16)

        @pl.when(wid == 0)
        def _():
            pltpu.sync_copy(red_s, sh_s)
            pltpu.sync_copy(red_c, sh_c)
        plsc.subcore_barrier()

        pltpu.sync_copy(lab_hbm.at[pl.ds(wid * EPW, EPW)], lab_v)
        pltpu.sync_copy(loss_hbm.at[pl.ds(wid * EPW, EPW)], loss_v)

        lane_base = lane * CPAD

        def acc_body(i, _):
            lab = lab_v[pl.ds(i * 16, 16)]
            lv = loss_v[pl.ds(i * 16, 16)]
            idx = lane_base + lab
            plsc.addupdate_scatter(bins, [idx], lv)
            plsc.addupdate_scatter(cbins, [idx], ones)
            return 0
        lax.fori_loop(0, EPW // 16, acc_body, 0)

        for j in range(RROWS):
            acc_s = bins[pl.ds(j * 16, 16)]
            acc_c = cbins[pl.ds(j * 16, 16)]
            for l in range(1, 16):
                acc_s = acc_s + bins[pl.ds(l * CPAD + j * 16, 16)]
                acc_c = acc_c + cbins[pl.ds(l * CPAD + j * 16, 16)]
            red_s[j, :] = acc_s
            red_c[j, :] = acc_c

        pltpu.sync_copy(red_s, sh_s.at[idx_v], add=True)
        pltpu.sync_copy(red_c, sh_c.at[idx_v], add=True)
        plsc.subcore_barrier()

        @pl.when(wid == 0)
        def _():
            pltpu.sync_copy(sh_s, sum_hbm)
            pltpu.sync_copy(sh_c, cnt_hbm)

    return k(losses, labels)


def kernel(inputs, labels):
    labels = labels.astype(jnp.int32)
    losses = _losses(inputs, labels)
    sums, counts = _groupby(losses, labels)
    return sums.reshape(-1)[:C], counts.reshape(-1)[:C]


# speedup vs baseline: 1.2256x; 1.2053x over previous
import functools

import jax
import jax.numpy as jnp
from jax import lax
from jax.experimental import pallas as pl
from jax.experimental.pallas import tpu as pltpu
from jax.experimental.pallas import tpu_sc as plsc

N = 16384
C = 1000
CPAD = 1024
BR = 1024
NW = 16
EPW = N // NW
RROWS = CPAD // 16


def _losses_body(x_ref, lab_ref, out_ref):
    x = x_ref[...]
    lab = lab_ref[...]
    m = jnp.max(x, axis=1)
    e = jnp.exp(x - m[:, None])
    s = jnp.sum(e, axis=1)
    logz = m + jnp.log(s)
    col = lax.broadcasted_iota(jnp.int32, x.shape, 1)
    picked = jnp.sum(jnp.where(col == lab[:, None], x, 0.0), axis=1)
    out_ref[...] = logz - picked


def _losses(inputs, labels):
    return pl.pallas_call(
        _losses_body,
        grid=(N // BR,),
        in_specs=[
            pl.BlockSpec((BR, C), lambda i: (i, 0)),
            pl.BlockSpec((BR,), lambda i: (i,)),
        ],
        out_specs=pl.BlockSpec((BR,), lambda i: (i,)),
        out_shape=jax.ShapeDtypeStruct((N,), jnp.float32),
    )(inputs, labels)


def _groupby(losses, labels):
    mesh = plsc.VectorSubcoreMesh(
        core_axis_name="c", subcore_axis_name="s", num_cores=1)

    @functools.partial(
        pl.kernel,
        mesh=mesh,
        compiler_params=pltpu.CompilerParams(
            use_tc_tiling_on_sc=False, needs_layout_passes=False),
        out_type=[
            jax.ShapeDtypeStruct((RROWS, 16), jnp.float32),
            jax.ShapeDtypeStruct((RROWS, 16), jnp.float32),
        ],
        scratch_types=[
            pltpu.VMEM((EPW,), jnp.int32),
            pltpu.VMEM((EPW,), jnp.float32),
            pltpu.VMEM((16 * CPAD,), jnp.float32),
            pltpu.VMEM((16 * CPAD,), jnp.float32),
            pltpu.VMEM((RROWS, 16), jnp.float32),
            pltpu.VMEM((RROWS, 16), jnp.float32),
            pltpu.VMEM((RROWS,), jnp.int32),
            pltpu.VMEM_SHARED((RROWS, 16), jnp.float32),
            pltpu.VMEM_SHARED((RROWS, 16), jnp.float32),
        ],
    )
    def k(loss_hbm, lab_hbm, sum_hbm, cnt_hbm,
          lab_v, loss_v, bins, cbins, red_s, red_c, idx_v, sh_s, sh_c):
        wid = lax.axis_index("s")
        lane = lax.broadcasted_iota(jnp.int32, (16,), 0)
        zeros = jnp.zeros((16,), jnp.float32)
        ones = jnp.ones((16,), jnp.float32)

        def zero_body(j, _):
            bins[pl.ds(j * 16, 16)] = zeros
            cbins[pl.ds(j * 16, 16)] = zeros
            return 0
        lax.fori_loop(0, 16 * CPAD // 16, zero_body, 0)
        for j in range(RROWS):
            red_s[j, :] = zeros
            red_c[j, :] = zeros
        for j in range(RROWS // 16):
            idx_v[pl.ds(j * 16, 16)] = lane + (j * 16)

        @pl.when(wid == 0)
        def _():
            pltpu.sync_copy(red_s, sh_s)
            pltpu.sync_copy(red_c, sh_c)
        plsc.subcore_barrier()

        pltpu.sync_copy(lab_hbm.at[pl.ds(wid * EPW, EPW)], lab_v)
        pltpu.sync_copy(loss_hbm.at[pl.ds(wid * EPW, EPW)], loss_v)

        lane_base = lane * CPAD

        def acc_body(i, _):
            lab = lab_v[pl.ds(i * 16, 16)]
            lv = loss_v[pl.ds(i * 16, 16)]
            idx = lane_base + lab
            plsc.addupdate_scatter(bins, [idx], lv)
            plsc.addupdate_scatter(cbins, [idx], ones)
            return 0
        lax.fori_loop(0, EPW // 16, acc_body, 0)

        for j in range(RROWS):
            acc_s = bins[pl.ds(j * 16, 16)]
            acc_c = cbins[pl.ds(j * 16, 16)]
            for l in range(1, 16):
                acc_s = acc_s + bins[pl.ds(l * CPAD + j * 16, 16)]
                acc_c = acc_c + cbins[pl.ds(l * CPAD + j * 16, 16)]
            red_s[j, :] = acc_s
            red_c[j, :] = acc_c

        pltpu.sync_copy(red_s, sh_s.at[idx_v], add=True)
        pltpu.sync_copy(red_c, sh_c.at[idx_v], add=True)
        plsc.subcore_barrier()

        @pl.when(wid == 0)
        def _():
            pltpu.sync_copy(sh_s, sum_hbm)
            pltpu.sync_copy(sh_c, cnt_hbm)

    return k(losses, labels)


def kernel(inputs, labels):
    labels = labels.astype(jnp.int32)
    losses = _losses(inputs, labels)
    sums, counts = _groupby(losses, labels)
    return sums.reshape(-1)[:C], counts.reshape(-1)[:C]


# speedup vs baseline: 1.2744x; 1.0399x over previous
import functools

import jax
import jax.numpy as jnp
from jax import lax
from jax.experimental import pallas as pl
from jax.experimental.pallas import tpu as pltpu
from jax.experimental.pallas import tpu_sc as plsc

N = 16384
C = 1000
CPAD = 1024
BR = 2048
NW = 16
EPW = N // NW
RROWS = CPAD // 16


def _losses_body(x_ref, lab_ref, out_ref):
    x = x_ref[...]
    lab = lab_ref[...]
    m = jnp.max(x, axis=1)
    e = jnp.exp(x - m[:, None])
    s = jnp.sum(e, axis=1)
    logz = m + jnp.log(s)
    col = lax.broadcasted_iota(jnp.int32, x.shape, 1)
    picked = jnp.sum(jnp.where(col == lab[:, None], x, 0.0), axis=1)
    out_ref[...] = logz - picked


def _losses(inputs, labels):
    return pl.pallas_call(
        _losses_body,
        grid=(N // BR,),
        in_specs=[
            pl.BlockSpec((BR, C), lambda i: (i, 0)),
            pl.BlockSpec((BR,), lambda i: (i,)),
        ],
        out_specs=pl.BlockSpec((BR,), lambda i: (i,)),
        out_shape=jax.ShapeDtypeStruct((N,), jnp.float32),
    )(inputs, labels)


def _groupby(losses, labels):
    mesh = plsc.VectorSubcoreMesh(
        core_axis_name="c", subcore_axis_name="s", num_cores=1)

    @functools.partial(
        pl.kernel,
        mesh=mesh,
        compiler_params=pltpu.CompilerParams(
            use_tc_tiling_on_sc=False, needs_layout_passes=False),
        out_type=[
            jax.ShapeDtypeStruct((RROWS, 16), jnp.float32),
            jax.ShapeDtypeStruct((RROWS, 16), jnp.float32),
        ],
        scratch_types=[
            pltpu.VMEM((EPW,), jnp.int32),
            pltpu.VMEM((EPW,), jnp.float32),
            pltpu.VMEM((16 * CPAD,), jnp.float32),
            pltpu.VMEM((16 * CPAD,), jnp.float32),
            pltpu.VMEM((RROWS, 16), jnp.float32),
            pltpu.VMEM((RROWS, 16), jnp.float32),
            pltpu.VMEM((RROWS,), jnp.int32),
            pltpu.VMEM_SHARED((RROWS, 16), jnp.float32),
            pltpu.VMEM_SHARED((RROWS, 16), jnp.float32),
        ],
    )
    def k(loss_hbm, lab_hbm, sum_hbm, cnt_hbm,
          lab_v, loss_v, bins, cbins, red_s, red_c, idx_v, sh_s, sh_c):
        wid = lax.axis_index("s")
        lane = lax.broadcasted_iota(jnp.int32, (16,), 0)
        zeros = jnp.zeros((16,), jnp.float32)
        ones = jnp.ones((16,), jnp.float32)

        def zero_body(j, _):
            bins[pl.ds(j * 16, 16)] = zeros
            cbins[pl.ds(j * 16, 16)] = zeros
            return 0
        lax.fori_loop(0, 16 * CPAD // 16, zero_body, 0)
        for j in range(RROWS):
            red_s[j, :] = zeros
            red_c[j, :] = zeros
        for j in range(RROWS // 16):
            idx_v[pl.ds(j * 16, 16)] = lane + (j * 16)

        @pl.when(wid == 0)
        def _():
            pltpu.sync_copy(red_s, sh_s)
            pltpu.sync_copy(red_c, sh_c)
        plsc.subcore_barrier()

        pltpu.sync_copy(lab_hbm.at[pl.ds(wid * EPW, EPW)], lab_v)
        pltpu.sync_copy(loss_hbm.at[pl.ds(wid * EPW, EPW)], loss_v)

        lane_base = lane * CPAD

        def acc_body(i, _):
            lab = lab_v[pl.ds(i * 16, 16)]
            lv = loss_v[pl.ds(i * 16, 16)]
            idx = lane_base + lab
            plsc.addupdate_scatter(bins, [idx], lv)
            plsc.addupdate_scatter(cbins, [idx], ones)
            return 0
        lax.fori_loop(0, EPW // 16, acc_body, 0)

        for j in range(RROWS):
            acc_s = bins[pl.ds(j * 16, 16)]
            acc_c = cbins[pl.ds(j * 16, 16)]
            for l in range(1, 16):
                acc_s = acc_s + bins[pl.ds(l * CPAD + j * 16, 16)]
                acc_c = acc_c + cbins[pl.ds(l * CPAD + j * 16, 16)]
            red_s[j, :] = acc_s
            red_c[j, :] = acc_c

        pltpu.sync_copy(red_s, sh_s.at[idx_v], add=True)
        pltpu.sync_copy(red_c, sh_c.at[idx_v], add=True)
        plsc.subcore_barrier()

        @pl.when(wid == 0)
        def _():
            pltpu.sync_copy(sh_s, sum_hbm)
            pltpu.sync_copy(sh_c, cnt_hbm)

    return k(losses, labels)


def kernel(inputs, labels):
    labels = labels.astype(jnp.int32)
    losses = _losses(inputs, labels)
    sums, counts = _groupby(losses, labels)
    return sums.reshape(-1)[:C], counts.reshape(-1)[:C]


# speedup vs baseline: 3.0092x; 2.3611x over previous
import functools

import jax
import jax.numpy as jnp
from jax import lax
from jax.experimental import pallas as pl
from jax.experimental.pallas import tpu as pltpu
from jax.experimental.pallas import tpu_sc as plsc

N = 16384
C = 1000
CPAD = 1024
BR = 4096
NW = 16
EPW = N // NW
RROWS = CPAD // 16


def _losses_body(x_ref, lab_ref, out_ref):
    x = x_ref[...]
    lab = lab_ref[...]
    m = jnp.max(x, axis=1)
    e = jnp.exp(x - m[:, None])
    s = jnp.sum(e, axis=1)
    logz = m + jnp.log(s)
    col = lax.broadcasted_iota(jnp.int32, x.shape, 1)
    picked = jnp.sum(jnp.where(col == lab[:, None], x, 0.0), axis=1)
    out_ref[...] = logz - picked


def _losses(inputs, labels):
    return pl.pallas_call(
        _losses_body,
        grid=(N // BR,),
        in_specs=[
            pl.BlockSpec((BR, C), lambda i: (i, 0)),
            pl.BlockSpec((BR,), lambda i: (i,)),
        ],
        out_specs=pl.BlockSpec((BR,), lambda i: (i,)),
        out_shape=jax.ShapeDtypeStruct((N,), jnp.float32),
    )(inputs, labels)


def _groupby(losses, labels):
    mesh = plsc.VectorSubcoreMesh(
        core_axis_name="c", subcore_axis_name="s", num_cores=1)

    @functools.partial(
        pl.kernel,
        mesh=mesh,
        compiler_params=pltpu.CompilerParams(
            use_tc_tiling_on_sc=False, needs_layout_passes=False),
        out_type=[
            jax.ShapeDtypeStruct((RROWS, 16), jnp.float32),
            jax.ShapeDtypeStruct((RROWS, 16), jnp.float32),
        ],
        scratch_types=[
            pltpu.VMEM((EPW,), jnp.int32),
            pltpu.VMEM((EPW,), jnp.float32),
            pltpu.VMEM((16 * CPAD,), jnp.float32),
            pltpu.VMEM((16 * CPAD,), jnp.float32),
            pltpu.VMEM((RROWS, 16), jnp.float32),
            pltpu.VMEM((RROWS, 16), jnp.float32),
            pltpu.VMEM((RROWS,), jnp.int32),
            pltpu.VMEM_SHARED((RROWS, 16), jnp.float32),
            pltpu.VMEM_SHARED((RROWS, 16), jnp.float32),
        ],
    )
    def k(loss_hbm, lab_hbm, sum_hbm, cnt_hbm,
          lab_v, loss_v, bins, cbins, red_s, red_c, idx_v, sh_s, sh_c):
        wid = lax.axis_index("s")
        lane = lax.broadcasted_iota(jnp.int32, (16,), 0)
        zeros = jnp.zeros((16,), jnp.float32)
        ones = jnp.ones((16,), jnp.float32)

        def zero_body(j, _):
            for u in range(16):
                bins[pl.ds(j * 256 + u * 16, 16)] = zeros
                cbins[pl.ds(j * 256 + u * 16, 16)] = zeros
            return 0
        lax.fori_loop(0, 16 * CPAD // 256, zero_body, 0)
        for j in range(RROWS):
            red_s[j, :] = zeros
            red_c[j, :] = zeros
        for j in range(RROWS // 16):
            idx_v[pl.ds(j * 16, 16)] = lane + (j * 16)

        @pl.when(wid == 0)
        def _():
            pltpu.sync_copy(red_s, sh_s)
            pltpu.sync_copy(red_c, sh_c)
        plsc.subcore_barrier()

        pltpu.sync_copy(lab_hbm.at[pl.ds(wid * EPW, EPW)], lab_v)
        pltpu.sync_copy(loss_hbm.at[pl.ds(wid * EPW, EPW)], loss_v)

        lane_base = lane * CPAD

        def acc_body(i, _):
            for u in range(8):
                lab = lab_v[pl.ds(i * 128 + u * 16, 16)]
                lv = loss_v[pl.ds(i * 128 + u * 16, 16)]
                idx = lane_base + lab
                plsc.addupdate_scatter(bins, [idx], lv)
                plsc.addupdate_scatter(cbins, [idx], ones)
            return 0
        lax.fori_loop(0, EPW // 128, acc_body, 0)

        for j in range(RROWS):
            acc_s = bins[pl.ds(j * 16, 16)]
            acc_c = cbins[pl.ds(j * 16, 16)]
            for l in range(1, 16):
                acc_s = acc_s + bins[pl.ds(l * CPAD + j * 16, 16)]
                acc_c = acc_c + cbins[pl.ds(l * CPAD + j * 16, 16)]
            red_s[j, :] = acc_s
            red_c[j, :] = acc_c

        pltpu.sync_copy(red_s, sh_s.at[idx_v], add=True)
        pltpu.sync_copy(red_c, sh_c.at[idx_v], add=True)
        plsc.subcore_barrier()

        @pl.when(wid == 0)
        def _():
            pltpu.sync_copy(sh_s, sum_hbm)
            pltpu.sync_copy(sh_c, cnt_hbm)

    return k(losses, labels)


def kernel(inputs, labels):
    labels = labels.astype(jnp.int32)
    losses = _losses(inputs, labels)
    sums, counts = _groupby(losses, labels)
    return sums.reshape(-1)[:C], counts.reshape(-1)[:C]
